# Initial kernel scaffold; baseline (speedup 1.0000x reference)
import jax
import jax.numpy as jnp
from jax.experimental import pallas as pl


def kernel(x, edge_index, W0, b0, W1, b1, p, Wu, bu):
    raise NotImplementedError("write your pallas kernel here")


# speedup vs baseline: 8.7779x; 8.7779x over previous
import functools
import math

import jax
import jax.numpy as jnp
from jax import lax
from jax.experimental import pallas as pl
from jax.experimental.pallas import tpu as pltpu
from jax.experimental.pallas import tpu_sc as plsc

N = 10000
D = 128
E = 160000
KSEL = 5000

NPAD = 10240
PHANTOM = N
NW = 32
EPT = 5120
G = 128
NCHUNK = EPT // G
NSUB = 16
RPT = NPAD // NSUB
BR = 1280


@functools.lru_cache(maxsize=1)
def _make_spmv_sc():
    mesh = plsc.VectorSubcoreMesh(core_axis_name="c", subcore_axis_name="s")

    @functools.partial(
        pl.kernel,
        mesh=mesh,
        out_type=jax.ShapeDtypeStruct((2 * NPAD, D), jnp.float32),
        scratch_types=[
            pltpu.VMEM((NCHUNK, G), jnp.int32),
            pltpu.VMEM((NCHUNK, G), jnp.int32),
            pltpu.VMEM((G, D), jnp.float32),
            pltpu.VMEM_SHARED((NPAD, D), jnp.float32),
            pltpu.SemaphoreType.DMA,
        ],
    )
    def _spmv_sc(h_hbm, rows_hbm, cols_hbm, zeros_hbm, out_hbm,
                 rows_v, cols_v, gbuf, acc, sem):
        cid = lax.axis_index("c")
        sid = lax.axis_index("s")
        wid = sid * 2 + cid

        pltpu.sync_copy(rows_hbm.at[wid], rows_v)
        pltpu.sync_copy(cols_hbm.at[wid], cols_v)
        pltpu.sync_copy(zeros_hbm.at[pl.ds(sid * RPT, RPT)],
                        acc.at[pl.ds(sid * RPT, RPT)])
        plsc.subcore_barrier()

        def chunk(i, carry):
            pltpu.async_copy(h_hbm.at[rows_v.at[i]], gbuf, sem).wait()
            pltpu.sync_copy(gbuf, acc.at[cols_v.at[i]], add=True)
            return carry

        lax.fori_loop(0, NCHUNK, chunk, 0)
        plsc.subcore_barrier()

        pltpu.sync_copy(acc.at[pl.ds(sid * RPT, RPT)],
                        out_hbm.at[pl.ds(cid * NPAD + sid * RPT, RPT)])

    return _spmv_sc


def _spmv(h, rows_sc, cols_sc, zeros_h):
    out = _make_spmv_sc()(h, rows_sc, cols_sc, zeros_h)
    return out[:NPAD], out[NPAD:]


def _mm_body(s_ref, x_ref, w_ref, o_ref):
    xs = s_ref[...] * x_ref[...]
    o_ref[...] = lax.dot_general(
        xs, w_ref[...], (((1,), (0,)), ((), ())),
        precision=lax.Precision.HIGHEST,
        preferred_element_type=jnp.float32)


def _scaled_mm(x, w, s):
    grid = NPAD // BR
    return pl.pallas_call(
        _mm_body,
        grid=(grid,),
        in_specs=[
            pl.BlockSpec((BR, 1), lambda i: (i, 0)),
            pl.BlockSpec((BR, D), lambda i: (i, 0)),
            pl.BlockSpec((D, D), lambda i: (0, 0)),
        ],
        out_specs=pl.BlockSpec((BR, D), lambda i: (i, 0)),
        out_shape=jax.ShapeDtypeStruct((NPAD, D), jnp.float32),
    )(s, x, w)


def _x1_score_body(v1_ref, v2_ref, p0_ref, p1_ref, h_ref, b_ref, pn_ref,
                   x1_ref, sc_ref):
    x1 = jax.nn.relu(v1_ref[...] * (p0_ref[...] + p1_ref[...])
                     + v2_ref[...] * h_ref[...] + b_ref[...])
    x1_ref[...] = x1
    sc_ref[...] = jnp.tanh(
        lax.dot_general(x1, pn_ref[...], (((1,), (0,)), ((), ())),
                        precision=lax.Precision.HIGHEST,
                        preferred_element_type=jnp.float32))


def _x1_score(v1, v2, p0, p1, h0s, b0row, pn):
    grid = NPAD // BR
    vec = pl.BlockSpec((BR, 1), lambda i: (i, 0))
    big = pl.BlockSpec((BR, D), lambda i: (i, 0))
    return pl.pallas_call(
        _x1_score_body,
        grid=(grid,),
        in_specs=[vec, vec, big, big, big,
                  pl.BlockSpec((1, D), lambda i: (0, 0)),
                  pl.BlockSpec((D, 1), lambda i: (0, 0))],
        out_specs=[big, vec],
        out_shape=[jax.ShapeDtypeStruct((NPAD, D), jnp.float32),
                   jax.ShapeDtypeStruct((NPAD, 1), jnp.float32)],
    )(v1, v2, p0, p1, h0s, b0row, pn)


def _t1_body(a_ref, y_ref, t0_ref, t1_ref, o_ref):
    o_ref[...] = a_ref[...] * y_ref[...] + t0_ref[...] + t1_ref[...]


def _t1_combine(a, y, t0, t1):
    grid = NPAD // BR
    vec = pl.BlockSpec((BR, 1), lambda i: (i, 0))
    big = pl.BlockSpec((BR, D), lambda i: (i, 0))
    return pl.pallas_call(
        _t1_body,
        grid=(grid,),
        in_specs=[vec, big, big, big],
        out_specs=big,
        out_shape=jax.ShapeDtypeStruct((NPAD, D), jnp.float32),
    )(a, y, t0, t1)


def _x3_body(a_ref, d2_ref, g_ref, u_ref, b_ref, t1_ref, q0_ref, q1_ref,
             y_ref, x1_ref, o_ref):
    t2 = a_ref[...] * t1_ref[...] + q0_ref[...] + q1_ref[...]
    z = t2 - d2_ref[...] * y_ref[...]
    x2 = jax.nn.relu(g_ref[...] * z + 2.0 * g_ref[...] * y_ref[...]
                     + b_ref[...])
    o_ref[...] = x1_ref[...] + u_ref[...] * x2


def _x3_combine(a, d2v, g1v, uv, b1row, t1, q0, q1, y, x1):
    grid = NPAD // BR
    vec = pl.BlockSpec((BR, 1), lambda i: (i, 0))
    big = pl.BlockSpec((BR, D), lambda i: (i, 0))
    return pl.pallas_call(
        _x3_body,
        grid=(grid,),
        in_specs=[vec, vec, vec, vec,
                  pl.BlockSpec((1, D), lambda i: (0, 0)),
                  big, big, big, big, big],
        out_specs=big,
        out_shape=jax.ShapeDtypeStruct((NPAD, D), jnp.float32),
    )(a, d2v, g1v, uv, b1row, t1, q0, q1, y, x1)


def _out_body(v1_ref, v2_ref, r0_ref, r1_ref, h_ref, b_ref, o_ref):
    o_ref[...] = (v1_ref[...] * (r0_ref[...] + r1_ref[...])
                  + v2_ref[...] * h_ref[...] + b_ref[...])


def _out_combine(v1, v2, r0, r1, hus, burow):
    grid = NPAD // BR
    vec = pl.BlockSpec((BR, 1), lambda i: (i, 0))
    big = pl.BlockSpec((BR, D), lambda i: (i, 0))
    return pl.pallas_call(
        _out_body,
        grid=(grid,),
        in_specs=[vec, vec, big, big, big,
                  pl.BlockSpec((1, D), lambda i: (0, 0))],
        out_specs=big,
        out_shape=jax.ShapeDtypeStruct((NPAD, D), jnp.float32),
    )(v1, v2, r0, r1, hus, burow)


def _pad1(v, fill=0.0):
    return jnp.pad(v, (0, NPAD - N), constant_values=fill).reshape(NPAD, 1)


def kernel(x, edge_index, W0, b0, W1, b1, p, Wu, bu):
    row = edge_index[0].astype(jnp.int32)
    col = edge_index[1].astype(jnp.int32)
    is_self = row == col
    ones_e = jnp.ones((E,), jnp.float32)

    cnt_all = jax.ops.segment_sum(ones_e, col, num_segments=N)
    cnt_self = jax.ops.segment_sum(is_self.astype(jnp.float32), col,
                                   num_segments=N)
    loop_w = jnp.where(cnt_self > 0, 0.0, 2.0)
    deg0 = cnt_all + loop_w
    dinv = 1.0 / jnp.sqrt(deg0)

    key = row * N + col
    ks = jnp.sort(key)
    rk = col * N + row
    hi = jnp.searchsorted(ks, rk, side="right")
    lo = jnp.searchsorted(ks, rk, side="left")
    revcnt = (hi - lo).astype(jnp.float32)
    d2 = 1.0 + jax.ops.segment_sum(jnp.where(is_self, 0.0, revcnt), row,
                                   num_segments=N)

    pad_e = NW * EPT - E
    rows_sc = jnp.concatenate(
        [row, jnp.full((pad_e,), PHANTOM, jnp.int32)]).reshape(NW, NCHUNK, G)
    cols_sc = jnp.concatenate(
        [col, jnp.full((pad_e,), PHANTOM, jnp.int32)]).reshape(NW, NCHUNK, G)
    zeros_h = jnp.zeros((NPAD, D), jnp.float32)

    x_pad = jnp.pad(x, ((0, NPAD - N), (0, 0)))
    dinv_p = _pad1(dinv)
    dlw_p = _pad1(dinv * loop_w)
    b0row = b0.reshape(1, D)
    b1row = b1.reshape(1, D)
    burow = bu.reshape(1, D)
    pn = (p / jnp.linalg.norm(p)).reshape(D, 1)

    h0s = _scaled_mm(x_pad, W0, dinv_p)
    p0, p1 = _spmv(h0s, rows_sc, cols_sc, zeros_h)
    x1, score = _x1_score(dinv_p, dlw_p, p0, p1, h0s, b0row, pn)

    score1d = score[:N, 0]
    _, perm = lax.top_k(score1d, KSEL)
    u = jnp.zeros((N,), jnp.float32).at[perm].set(1.0)

    nonself = jnp.logical_not(is_self)
    s1 = u + jax.ops.segment_sum(jnp.where(nonself, u[row], 0.0), col,
                                 num_segments=N)
    s2 = s1 + jax.ops.segment_sum(jnp.where(nonself, s1[row], 0.0), col,
                                  num_segments=N)
    deg1 = s2 - d2 * u + 2.0
    dinv1m = jnp.where(u > 0, 1.0 / jnp.sqrt(deg1), 0.0)

    sc1 = _pad1(dinv1m * score1d)
    y = _scaled_mm(x1, W1, sc1)
    t0a, t0b = _spmv(y, rows_sc, cols_sc, zeros_h)
    a_p = _pad1(1.0 - cnt_self, fill=1.0)
    t1 = _t1_combine(a_p, y, t0a, t0b)
    q0, q1 = _spmv(t1, rows_sc, cols_sc, zeros_h)
    x3 = _x3_combine(a_p, _pad1(d2), _pad1(dinv1m), _pad1(u), b1row,
                     t1, q0, q1, y, x1)

    hus = _scaled_mm(x3, Wu, dinv_p)
    r0, r1 = _spmv(hus, rows_sc, cols_sc, zeros_h)
    out = _out_combine(dinv_p, dlw_p, r0, r1, hus, burow)
    return out[:N]


# speedup vs baseline: 12.0835x; 1.3766x over previous
import functools
import math

import jax
import jax.numpy as jnp
from jax import lax
from jax.experimental import pallas as pl
from jax.experimental.pallas import tpu as pltpu
from jax.experimental.pallas import tpu_sc as plsc

N = 10000
D = 128
E = 160000
KSEL = 5000

NPAD = 10240
PHANTOM = N
NW = 32
EPT = 5120
G = 128
NCHUNK = EPT // G
NSUB = 16
RPT = NPAD // NSUB
BR = 1280


@functools.lru_cache(maxsize=1)
def _make_spmv_sc():
    mesh = plsc.VectorSubcoreMesh(core_axis_name="c", subcore_axis_name="s")

    @functools.partial(
        pl.kernel,
        mesh=mesh,
        out_type=jax.ShapeDtypeStruct((2 * NPAD, D), jnp.float32),
        scratch_types=[
            pltpu.VMEM((NCHUNK, G), jnp.int32),
            pltpu.VMEM((NCHUNK, G), jnp.int32),
            pltpu.VMEM((G, D), jnp.float32),
            pltpu.VMEM((G, D), jnp.float32),
            pltpu.VMEM_SHARED((NPAD, D), jnp.float32),
            pltpu.SemaphoreType.DMA,
            pltpu.SemaphoreType.DMA,
        ],
    )
    def _spmv_sc(h_hbm, rows_hbm, cols_hbm, zeros_hbm, out_hbm,
                 rows_v, cols_v, gbuf, gbuf2, acc, semA, semB):
        cid = lax.axis_index("c")
        sid = lax.axis_index("s")
        wid = sid * 2 + cid

        pltpu.sync_copy(rows_hbm.at[wid], rows_v)
        pltpu.sync_copy(cols_hbm.at[wid], cols_v)
        pltpu.sync_copy(zeros_hbm.at[pl.ds(sid * RPT, RPT)],
                        acc.at[pl.ds(sid * RPT, RPT)])
        plsc.subcore_barrier()

        pltpu.async_copy(h_hbm.at[rows_v.at[0]], gbuf, semA)

        def chunk(j, carry):
            i0 = 2 * j
            pltpu.async_copy(h_hbm.at[rows_v.at[i0 + 1]], gbuf2, semB)
            pltpu.make_async_copy(h_hbm.at[rows_v.at[i0]], gbuf, semA).wait()
            pltpu.sync_copy(gbuf, acc.at[cols_v.at[i0]], add=True)

            @pl.when(j < NCHUNK // 2 - 1)
            def _():
                pltpu.async_copy(h_hbm.at[rows_v.at[i0 + 2]], gbuf, semA)

            pltpu.make_async_copy(h_hbm.at[rows_v.at[i0 + 1]], gbuf2,
                                  semB).wait()
            pltpu.sync_copy(gbuf2, acc.at[cols_v.at[i0 + 1]], add=True)
            return carry

        lax.fori_loop(0, NCHUNK // 2, chunk, 0)
        plsc.subcore_barrier()

        pltpu.sync_copy(acc.at[pl.ds(sid * RPT, RPT)],
                        out_hbm.at[pl.ds(cid * NPAD + sid * RPT, RPT)])

    return _spmv_sc


def _spmv(h, rows_sc, cols_sc, zeros_h):
    out = _make_spmv_sc()(h, rows_sc, cols_sc, zeros_h)
    return out[:NPAD], out[NPAD:]


W8 = 8


@functools.lru_cache(maxsize=1)
def _make_scal_spmv_sc():
    mesh = plsc.VectorSubcoreMesh(core_axis_name="c", subcore_axis_name="s")

    @functools.partial(
        pl.kernel,
        mesh=mesh,
        out_type=jax.ShapeDtypeStruct((2 * NPAD, W8), jnp.float32),
        scratch_types=[
            pltpu.VMEM((NCHUNK, G), jnp.int32),
            pltpu.VMEM((NCHUNK, G), jnp.int32),
            pltpu.VMEM((G, W8), jnp.float32),
            pltpu.VMEM((G, W8), jnp.float32),
            pltpu.VMEM_SHARED((NPAD, W8), jnp.float32),
            pltpu.SemaphoreType.DMA,
            pltpu.SemaphoreType.DMA,
        ],
        compiler_params=pltpu.CompilerParams(use_tc_tiling_on_sc=False),
    )
    def _scal_spmv_sc(v_hbm, rows_hbm, cols_hbm, zeros_hbm, out_hbm,
                      rows_v, cols_v, gbuf, gbuf2, acc, semA, semB):
        cid = lax.axis_index("c")
        sid = lax.axis_index("s")
        wid = sid * 2 + cid

        pltpu.sync_copy(rows_hbm.at[wid], rows_v)
        pltpu.sync_copy(cols_hbm.at[wid], cols_v)
        pltpu.sync_copy(zeros_hbm.at[pl.ds(sid * RPT, RPT)],
                        acc.at[pl.ds(sid * RPT, RPT)])
        plsc.subcore_barrier()

        pltpu.async_copy(v_hbm.at[rows_v.at[0]], gbuf, semA)

        def chunk(j, carry):
            i0 = 2 * j
            pltpu.async_copy(v_hbm.at[rows_v.at[i0 + 1]], gbuf2, semB)
            pltpu.make_async_copy(v_hbm.at[rows_v.at[i0]], gbuf, semA).wait()
            pltpu.sync_copy(gbuf, acc.at[cols_v.at[i0]], add=True)

            @pl.when(j < NCHUNK // 2 - 1)
            def _():
                pltpu.async_copy(v_hbm.at[rows_v.at[i0 + 2]], gbuf, semA)

            pltpu.make_async_copy(v_hbm.at[rows_v.at[i0 + 1]], gbuf2,
                                  semB).wait()
            pltpu.sync_copy(gbuf2, acc.at[cols_v.at[i0 + 1]], add=True)
            return carry

        lax.fori_loop(0, NCHUNK // 2, chunk, 0)
        plsc.subcore_barrier()
        pltpu.sync_copy(acc.at[pl.ds(sid * RPT, RPT)],
                        out_hbm.at[pl.ds(cid * NPAD + sid * RPT, RPT)])

    return _scal_spmv_sc


def _scal_spmv(v_pad, rows_sc, cols_sc, zeros2):
    v2 = jnp.broadcast_to(v_pad[:, None], (NPAD, W8))
    out = _make_scal_spmv_sc()(v2, rows_sc, cols_sc, zeros2)
    return out[:NPAD, 0] + out[NPAD:, 0]


@functools.lru_cache(maxsize=1)
def _make_edge_scat_sc():
    mesh = plsc.VectorSubcoreMesh(core_axis_name="c", subcore_axis_name="s")

    @functools.partial(
        pl.kernel,
        mesh=mesh,
        out_type=jax.ShapeDtypeStruct((2 * NPAD, W8), jnp.float32),
        scratch_types=[
            pltpu.VMEM((NCHUNK, G), jnp.int32),
            pltpu.VMEM((G, W8), jnp.float32),
            pltpu.VMEM((G, W8), jnp.float32),
            pltpu.VMEM_SHARED((NPAD, W8), jnp.float32),
            pltpu.SemaphoreType.DMA,
            pltpu.SemaphoreType.DMA,
        ],
        compiler_params=pltpu.CompilerParams(use_tc_tiling_on_sc=False),
    )
    def _edge_scat_sc(ev_hbm, idx_hbm, zeros_hbm, out_hbm,
                      idx_v, gbuf, gbuf2, acc, semA, semB):
        cid = lax.axis_index("c")
        sid = lax.axis_index("s")
        wid = sid * 2 + cid

        pltpu.sync_copy(idx_hbm.at[wid], idx_v)
        pltpu.sync_copy(zeros_hbm.at[pl.ds(sid * RPT, RPT)],
                        acc.at[pl.ds(sid * RPT, RPT)])
        plsc.subcore_barrier()

        pltpu.async_copy(ev_hbm.at[wid, 0], gbuf, semA)

        def chunk(j, carry):
            i0 = 2 * j
            pltpu.async_copy(ev_hbm.at[wid, i0 + 1], gbuf2, semB)
            pltpu.make_async_copy(ev_hbm.at[wid, i0], gbuf, semA).wait()
            pltpu.sync_copy(gbuf, acc.at[idx_v.at[i0]], add=True)

            @pl.when(j < NCHUNK // 2 - 1)
            def _():
                pltpu.async_copy(ev_hbm.at[wid, i0 + 2], gbuf, semA)

            pltpu.make_async_copy(ev_hbm.at[wid, i0 + 1], gbuf2,
                                  semB).wait()
            pltpu.sync_copy(gbuf2, acc.at[idx_v.at[i0 + 1]], add=True)
            return carry

        lax.fori_loop(0, NCHUNK // 2, chunk, 0)
        plsc.subcore_barrier()
        pltpu.sync_copy(acc.at[pl.ds(sid * RPT, RPT)],
                        out_hbm.at[pl.ds(cid * NPAD + sid * RPT, RPT)])

    return _edge_scat_sc


def _edge_scat(ev, idx_sc, zeros2):
    ev_pad = jnp.concatenate([ev, jnp.zeros((NW * EPT - E,), jnp.float32)])
    ev2 = jnp.pad(ev_pad[:, None], ((0, 0), (0, W8 - 1)))
    ev4 = ev2.reshape(NW, NCHUNK, G, W8)
    out = _make_edge_scat_sc()(ev4, idx_sc, zeros2)
    return out[:NPAD, 0] + out[NPAD:, 0]


TR = NPAD // D


def _topk_body(s_ref, u_ref):
    s = s_ref[...]
    i32 = lax.bitcast_convert_type(s, jnp.int32)
    key = jnp.where(i32 < 0, i32 ^ jnp.int32(0x7FFFFFFF), i32)
    ukey = lax.bitcast_convert_type(key, jnp.uint32) ^ jnp.uint32(0x80000000)
    rown = lax.broadcasted_iota(jnp.int32, (TR, D), 0)
    coln = lax.broadcasted_iota(jnp.int32, (TR, D), 1)
    valid = rown * D + coln < N
    ukey = jnp.where(valid, ukey, jnp.uint32(0))

    def bit(b, t):
        sh = jnp.uint32(31) - b.astype(jnp.uint32)
        cand = t | lax.shift_left(jnp.uint32(1), sh)
        cnt = jnp.sum((ukey >= cand).astype(jnp.int32))
        return jnp.where(cnt >= KSEL, cand, t)

    t = lax.fori_loop(0, 32, bit, jnp.uint32(0))
    gt = ukey > t
    eq = ukey == t
    eqf = eq.astype(jnp.float32)
    su = (lax.broadcasted_iota(jnp.int32, (D, D), 0)
          < lax.broadcasted_iota(jnp.int32, (D, D), 1)).astype(jnp.float32)
    within = lax.dot_general(eqf, su, (((1,), (0,)), ((), ())),
                             precision=lax.Precision.HIGHEST,
                             preferred_element_type=jnp.float32)
    sl = (lax.broadcasted_iota(jnp.int32, (TR, TR), 1)
          < lax.broadcasted_iota(jnp.int32, (TR, TR), 0)).astype(jnp.float32)
    rowsum = jnp.sum(eqf, axis=1, keepdims=True)
    rowpre = lax.dot_general(sl, rowsum, (((1,), (0,)), ((), ())),
                             precision=lax.Precision.HIGHEST,
                             preferred_element_type=jnp.float32)
    excl = rowpre + within
    needf = (KSEL - jnp.sum(gt.astype(jnp.int32))).astype(jnp.float32)
    u_ref[...] = jnp.where(gt | (eq & (excl < needf)), 1.0, 0.0)


def _topk_mask(score_pad):
    u2d = pl.pallas_call(
        _topk_body,
        out_shape=jax.ShapeDtypeStruct((TR, D), jnp.float32),
    )(score_pad.reshape(TR, D))
    return u2d.reshape(NPAD)


def _mm_body(s_ref, x_ref, w_ref, o_ref):
    xs = s_ref[...] * x_ref[...]
    o_ref[...] = lax.dot_general(
        xs, w_ref[...], (((1,), (0,)), ((), ())),
        precision=lax.Precision.HIGHEST,
        preferred_element_type=jnp.float32)


def _scaled_mm(x, w, s):
    grid = NPAD // BR
    return pl.pallas_call(
        _mm_body,
        grid=(grid,),
        in_specs=[
            pl.BlockSpec((BR, 1), lambda i: (i, 0)),
            pl.BlockSpec((BR, D), lambda i: (i, 0)),
            pl.BlockSpec((D, D), lambda i: (0, 0)),
        ],
        out_specs=pl.BlockSpec((BR, D), lambda i: (i, 0)),
        out_shape=jax.ShapeDtypeStruct((NPAD, D), jnp.float32),
    )(s, x, w)


def _x1_score_body(v1_ref, v2_ref, p0_ref, p1_ref, h_ref, b_ref, pn_ref,
                   x1_ref, sc_ref):
    x1 = jax.nn.relu(v1_ref[...] * (p0_ref[...] + p1_ref[...])
                     + v2_ref[...] * h_ref[...] + b_ref[...])
    x1_ref[...] = x1
    sc_ref[...] = jnp.tanh(
        lax.dot_general(x1, pn_ref[...], (((1,), (0,)), ((), ())),
                        precision=lax.Precision.HIGHEST,
                        preferred_element_type=jnp.float32))


def _x1_score(v1, v2, p0, p1, h0s, b0row, pn):
    grid = NPAD // BR
    vec = pl.BlockSpec((BR, 1), lambda i: (i, 0))
    big = pl.BlockSpec((BR, D), lambda i: (i, 0))
    return pl.pallas_call(
        _x1_score_body,
        grid=(grid,),
        in_specs=[vec, vec, big, big, big,
                  pl.BlockSpec((1, D), lambda i: (0, 0)),
                  pl.BlockSpec((D, 1), lambda i: (0, 0))],
        out_specs=[big, vec],
        out_shape=[jax.ShapeDtypeStruct((NPAD, D), jnp.float32),
                   jax.ShapeDtypeStruct((NPAD, 1), jnp.float32)],
    )(v1, v2, p0, p1, h0s, b0row, pn)


def _t1_body(a_ref, y_ref, t0_ref, t1_ref, o_ref):
    o_ref[...] = a_ref[...] * y_ref[...] + t0_ref[...] + t1_ref[...]


def _t1_combine(a, y, t0, t1):
    grid = NPAD // BR
    vec = pl.BlockSpec((BR, 1), lambda i: (i, 0))
    big = pl.BlockSpec((BR, D), lambda i: (i, 0))
    return pl.pallas_call(
        _t1_body,
        grid=(grid,),
        in_specs=[vec, big, big, big],
        out_specs=big,
        out_shape=jax.ShapeDtypeStruct((NPAD, D), jnp.float32),
    )(a, y, t0, t1)


def _x3_body(a_ref, d2_ref, g_ref, u_ref, b_ref, t1_ref, q0_ref, q1_ref,
             y_ref, x1_ref, o_ref):
    t2 = a_ref[...] * t1_ref[...] + q0_ref[...] + q1_ref[...]
    z = t2 - d2_ref[...] * y_ref[...]
    x2 = jax.nn.relu(g_ref[...] * z + 2.0 * g_ref[...] * y_ref[...]
                     + b_ref[...])
    o_ref[...] = x1_ref[...] + u_ref[...] * x2


def _x3_combine(a, d2v, g1v, uv, b1row, t1, q0, q1, y, x1):
    grid = NPAD // BR
    vec = pl.BlockSpec((BR, 1), lambda i: (i, 0))
    big = pl.BlockSpec((BR, D), lambda i: (i, 0))
    return pl.pallas_call(
        _x3_body,
        grid=(grid,),
        in_specs=[vec, vec, vec, vec,
                  pl.BlockSpec((1, D), lambda i: (0, 0)),
                  big, big, big, big, big],
        out_specs=big,
        out_shape=jax.ShapeDtypeStruct((NPAD, D), jnp.float32),
    )(a, d2v, g1v, uv, b1row, t1, q0, q1, y, x1)


def _out_body(v1_ref, v2_ref, r0_ref, r1_ref, h_ref, b_ref, o_ref):
    o_ref[...] = (v1_ref[...] * (r0_ref[...] + r1_ref[...])
                  + v2_ref[...] * h_ref[...] + b_ref[...])


def _out_combine(v1, v2, r0, r1, hus, burow):
    grid = NPAD // BR
    vec = pl.BlockSpec((BR, 1), lambda i: (i, 0))
    big = pl.BlockSpec((BR, D), lambda i: (i, 0))
    return pl.pallas_call(
        _out_body,
        grid=(grid,),
        in_specs=[vec, vec, big, big, big,
                  pl.BlockSpec((1, D), lambda i: (0, 0))],
        out_specs=big,
        out_shape=jax.ShapeDtypeStruct((NPAD, D), jnp.float32),
    )(v1, v2, r0, r1, hus, burow)


def _pad1(v, fill=0.0):
    return jnp.pad(v, (0, NPAD - N), constant_values=fill).reshape(NPAD, 1)


def kernel(x, edge_index, W0, b0, W1, b1, p, Wu, bu):
    row = edge_index[0].astype(jnp.int32)
    col = edge_index[1].astype(jnp.int32)

    pad_e = NW * EPT - E
    rows_flat = jnp.concatenate([row, jnp.full((pad_e,), PHANTOM, jnp.int32)])
    cols_flat = jnp.concatenate([col, jnp.full((pad_e,), PHANTOM, jnp.int32)])
    rows_sc = rows_flat.reshape(NW, NCHUNK, G)
    cols_sc = cols_flat.reshape(NW, NCHUNK, G)
    zeros_h = jnp.zeros((NPAD, D), jnp.float32)
    zeros2 = jnp.zeros((NPAD, W8), jnp.float32)
    is_self = (row == col).astype(jnp.float32)

    cnt_all = _edge_scat(jnp.ones((E,), jnp.float32), cols_sc, zeros2)[:N]
    cnt_self = _edge_scat(is_self, cols_sc, zeros2)[:N]
    deg0 = cnt_all + jnp.where(cnt_self > 0, 0.0, 2.0)
    dinv = 1.0 / jnp.sqrt(deg0)

    key = row * N + col
    ks = jnp.sort(key)
    rk = col * N + row
    hi = jnp.searchsorted(ks, rk, side="right")
    lo = jnp.searchsorted(ks, rk, side="left")
    revcnt = jnp.where(row == col, 0.0, (hi - lo).astype(jnp.float32))
    d2 = 1.0 + _edge_scat(revcnt, rows_sc, zeros2)[:N]

    x_pad = jnp.pad(x, ((0, NPAD - N), (0, 0)))
    loop_w = jnp.where(cnt_self > 0, 0.0, 2.0)
    dinv_p = _pad1(dinv)
    dlw_p = _pad1(dinv * loop_w)
    b0row = b0.reshape(1, D)
    b1row = b1.reshape(1, D)
    burow = bu.reshape(1, D)
    pn = (p / jnp.linalg.norm(p)).reshape(D, 1)

    h0s = _scaled_mm(x_pad, W0, dinv_p)
    p0, p1 = _spmv(h0s, rows_sc, cols_sc, zeros_h)
    x1, score = _x1_score(dinv_p, dlw_p, p0, p1, h0s, b0row, pn)

    u_pad = _topk_mask(score[:, 0])
    u = u_pad[:N]
    score1d = score[:N, 0]

    s1 = (1.0 - cnt_self) * u + _scal_spmv(u_pad, rows_sc, cols_sc,
                                           zeros2)[:N]
    s1_pad = jnp.pad(s1, (0, NPAD - N))
    s2 = (1.0 - cnt_self) * s1 + _scal_spmv(s1_pad, rows_sc, cols_sc,
                                            zeros2)[:N]
    deg1 = s2 - d2 * u + 2.0
    dinv1m = jnp.where(u > 0, 1.0 / jnp.sqrt(deg1), 0.0)

    sc1 = _pad1(dinv1m * score1d)
    y = _scaled_mm(x1, W1, sc1)
    t0a, t0b = _spmv(y, rows_sc, cols_sc, zeros_h)
    a_p = _pad1(1.0 - cnt_self, fill=1.0)
    t1 = _t1_combine(a_p, y, t0a, t0b)
    q0, q1 = _spmv(t1, rows_sc, cols_sc, zeros_h)
    x3 = _x3_combine(a_p, _pad1(d2), _pad1(dinv1m), _pad1(u), b1row,
                     t1, q0, q1, y, x1)

    hus = _scaled_mm(x3, Wu, dinv_p)
    r0, r1 = _spmv(hus, rows_sc, cols_sc, zeros_h)
    out = _out_combine(dinv_p, dlw_p, r0, r1, hus, burow)
    return out[:N]


# speedup vs baseline: 29.4214x; 2.4348x over previous
import functools
import math

import jax
import jax.numpy as jnp
from jax import lax
from jax.experimental import pallas as pl
from jax.experimental.pallas import tpu as pltpu
from jax.experimental.pallas import tpu_sc as plsc

N = 10000
D = 128
E = 160000
KSEL = 5000

NPAD = 10240
PHANTOM = N
NW = 32
EPT = 5120
G = 128
NCHUNK = EPT // G
NSUB = 16
RPT = NPAD // NSUB
BR = 1280


@functools.lru_cache(maxsize=1)
def _make_spmv_sc():
    mesh = plsc.VectorSubcoreMesh(core_axis_name="c", subcore_axis_name="s")

    @functools.partial(
        pl.kernel,
        mesh=mesh,
        out_type=jax.ShapeDtypeStruct((2 * NPAD, D), jnp.float32),
        scratch_types=[
            pltpu.VMEM((NCHUNK, G), jnp.int32),
            pltpu.VMEM((NCHUNK, G), jnp.int32),
            pltpu.VMEM((G, D), jnp.float32),
            pltpu.VMEM((G, D), jnp.float32),
            pltpu.VMEM_SHARED((NPAD, D), jnp.float32),
            pltpu.SemaphoreType.DMA,
            pltpu.SemaphoreType.DMA,
        ],
    )
    def _spmv_sc(h_hbm, rows_hbm, cols_hbm, zeros_hbm, out_hbm,
                 rows_v, cols_v, gbuf, gbuf2, acc, semA, semB):
        cid = lax.axis_index("c")
        sid = lax.axis_index("s")
        wid = sid * 2 + cid

        pltpu.sync_copy(rows_hbm.at[wid], rows_v)
        pltpu.sync_copy(cols_hbm.at[wid], cols_v)
        pltpu.sync_copy(zeros_hbm.at[pl.ds(sid * RPT, RPT)],
                        acc.at[pl.ds(sid * RPT, RPT)])
        plsc.subcore_barrier()

        pltpu.async_copy(h_hbm.at[rows_v.at[0]], gbuf, semA)

        def chunk(j, carry):
            i0 = 2 * j
            pltpu.async_copy(h_hbm.at[rows_v.at[i0 + 1]], gbuf2, semB)
            pltpu.make_async_copy(h_hbm.at[rows_v.at[i0]], gbuf, semA).wait()
            pltpu.sync_copy(gbuf, acc.at[cols_v.at[i0]], add=True)

            @pl.when(j < NCHUNK // 2 - 1)
            def _():
                pltpu.async_copy(h_hbm.at[rows_v.at[i0 + 2]], gbuf, semA)

            pltpu.make_async_copy(h_hbm.at[rows_v.at[i0 + 1]], gbuf2,
                                  semB).wait()
            pltpu.sync_copy(gbuf2, acc.at[cols_v.at[i0 + 1]], add=True)
            return carry

        lax.fori_loop(0, NCHUNK // 2, chunk, 0)
        plsc.subcore_barrier()

        pltpu.sync_copy(acc.at[pl.ds(sid * RPT, RPT)],
                        out_hbm.at[pl.ds(cid * NPAD + sid * RPT, RPT)])

    return _spmv_sc


def _spmv(h, rows_sc, cols_sc, zeros_h):
    out = _make_spmv_sc()(h, rows_sc, cols_sc, zeros_h)
    return out[:NPAD], out[NPAD:]


W8 = 8


@functools.lru_cache(maxsize=1)
def _make_scal_spmv_sc():
    mesh = plsc.VectorSubcoreMesh(core_axis_name="c", subcore_axis_name="s")

    @functools.partial(
        pl.kernel,
        mesh=mesh,
        out_type=jax.ShapeDtypeStruct((2 * NPAD, W8), jnp.float32),
        scratch_types=[
            pltpu.VMEM((NCHUNK, G), jnp.int32),
            pltpu.VMEM((NCHUNK, G), jnp.int32),
            pltpu.VMEM((G, W8), jnp.float32),
            pltpu.VMEM((G, W8), jnp.float32),
            pltpu.VMEM_SHARED((NPAD, W8), jnp.float32),
            pltpu.SemaphoreType.DMA,
            pltpu.SemaphoreType.DMA,
        ],
        compiler_params=pltpu.CompilerParams(use_tc_tiling_on_sc=False),
    )
    def _scal_spmv_sc(v_hbm, rows_hbm, cols_hbm, zeros_hbm, out_hbm,
                      rows_v, cols_v, gbuf, gbuf2, acc, semA, semB):
        cid = lax.axis_index("c")
        sid = lax.axis_index("s")
        wid = sid * 2 + cid

        pltpu.sync_copy(rows_hbm.at[wid], rows_v)
        pltpu.sync_copy(cols_hbm.at[wid], cols_v)
        pltpu.sync_copy(zeros_hbm.at[pl.ds(sid * RPT, RPT)],
                        acc.at[pl.ds(sid * RPT, RPT)])
        plsc.subcore_barrier()

        pltpu.async_copy(v_hbm.at[rows_v.at[0]], gbuf, semA)

        def chunk(j, carry):
            i0 = 2 * j
            pltpu.async_copy(v_hbm.at[rows_v.at[i0 + 1]], gbuf2, semB)
            pltpu.make_async_copy(v_hbm.at[rows_v.at[i0]], gbuf, semA).wait()
            pltpu.sync_copy(gbuf, acc.at[cols_v.at[i0]], add=True)

            @pl.when(j < NCHUNK // 2 - 1)
            def _():
                pltpu.async_copy(v_hbm.at[rows_v.at[i0 + 2]], gbuf, semA)

            pltpu.make_async_copy(v_hbm.at[rows_v.at[i0 + 1]], gbuf2,
                                  semB).wait()
            pltpu.sync_copy(gbuf2, acc.at[cols_v.at[i0 + 1]], add=True)
            return carry

        lax.fori_loop(0, NCHUNK // 2, chunk, 0)
        plsc.subcore_barrier()
        pltpu.sync_copy(acc.at[pl.ds(sid * RPT, RPT)],
                        out_hbm.at[pl.ds(cid * NPAD + sid * RPT, RPT)])

    return _scal_spmv_sc


def _scal_spmv(v_pad, rows_sc, cols_sc, zeros2):
    v2 = jnp.broadcast_to(v_pad[:, None], (NPAD, W8))
    out = _make_scal_spmv_sc()(v2, rows_sc, cols_sc, zeros2)
    return out[:NPAD, 0] + out[NPAD:, 0]


@functools.lru_cache(maxsize=None)
def _make_edge_scat_sc(nchunk):
    mesh = plsc.VectorSubcoreMesh(core_axis_name="c", subcore_axis_name="s")

    @functools.partial(
        pl.kernel,
        mesh=mesh,
        out_type=jax.ShapeDtypeStruct((2 * NPAD, W8), jnp.float32),
        scratch_types=[
            pltpu.VMEM((nchunk, G), jnp.int32),
            pltpu.VMEM((G, W8), jnp.float32),
            pltpu.VMEM((G, W8), jnp.float32),
            pltpu.VMEM_SHARED((NPAD, W8), jnp.float32),
            pltpu.SemaphoreType.DMA,
            pltpu.SemaphoreType.DMA,
        ],
        compiler_params=pltpu.CompilerParams(use_tc_tiling_on_sc=False),
    )
    def _edge_scat_sc(ev_hbm, idx_hbm, zeros_hbm, out_hbm,
                      idx_v, gbuf, gbuf2, acc, semA, semB):
        cid = lax.axis_index("c")
        sid = lax.axis_index("s")
        wid = sid * 2 + cid

        pltpu.sync_copy(idx_hbm.at[wid], idx_v)
        pltpu.sync_copy(zeros_hbm.at[pl.ds(sid * RPT, RPT)],
                        acc.at[pl.ds(sid * RPT, RPT)])
        plsc.subcore_barrier()

        pltpu.async_copy(ev_hbm.at[wid, 0], gbuf, semA)

        def chunk(j, carry):
            i0 = 2 * j
            pltpu.async_copy(ev_hbm.at[wid, i0 + 1], gbuf2, semB)
            pltpu.make_async_copy(ev_hbm.at[wid, i0], gbuf, semA).wait()
            pltpu.sync_copy(gbuf, acc.at[idx_v.at[i0]], add=True)

            @pl.when(j < nchunk // 2 - 1)
            def _():
                pltpu.async_copy(ev_hbm.at[wid, i0 + 2], gbuf, semA)

            pltpu.make_async_copy(ev_hbm.at[wid, i0 + 1], gbuf2,
                                  semB).wait()
            pltpu.sync_copy(gbuf2, acc.at[idx_v.at[i0 + 1]], add=True)
            return carry

        lax.fori_loop(0, nchunk // 2, chunk, 0)
        plsc.subcore_barrier()
        pltpu.sync_copy(acc.at[pl.ds(sid * RPT, RPT)],
                        out_hbm.at[pl.ds(cid * NPAD + sid * RPT, RPT)])

    return _edge_scat_sc


def _edge_scat(ev, idx, zeros2):
    L = ev.shape[0]
    nchunk = -(-L // (NW * G))
    nchunk += nchunk % 2
    Lp = NW * nchunk * G
    ev_pad = jnp.pad(ev, (0, Lp - L))
    idx_pad = jnp.pad(idx, (0, Lp - L), constant_values=PHANTOM)
    ev4 = jnp.pad(ev_pad[:, None], ((0, 0), (0, W8 - 1)))
    out = _make_edge_scat_sc(nchunk)(ev4.reshape(NW, nchunk, G, W8),
                                     idx_pad.reshape(NW, nchunk, G), zeros2)
    return out[:NPAD, 0] + out[NPAD:, 0]


TR = NPAD // D


def _topk_body(s_ref, u_ref):
    s = s_ref[...]
    i32 = lax.bitcast_convert_type(s, jnp.int32)
    key = jnp.where(i32 < 0, i32 ^ jnp.int32(0x7FFFFFFF), i32)
    ukey = lax.bitcast_convert_type(key, jnp.uint32) ^ jnp.uint32(0x80000000)
    rown = lax.broadcasted_iota(jnp.int32, (TR, D), 0)
    coln = lax.broadcasted_iota(jnp.int32, (TR, D), 1)
    valid = rown * D + coln < N
    ukey = jnp.where(valid, ukey, jnp.uint32(0))

    def bit(b, t):
        sh = jnp.uint32(31) - b.astype(jnp.uint32)
        cand = t | lax.shift_left(jnp.uint32(1), sh)
        cnt = jnp.sum((ukey >= cand).astype(jnp.int32))
        return jnp.where(cnt >= KSEL, cand, t)

    t = lax.fori_loop(0, 32, bit, jnp.uint32(0))
    gt = ukey > t
    eq = ukey == t
    eqf = eq.astype(jnp.float32)
    su = (lax.broadcasted_iota(jnp.int32, (D, D), 0)
          < lax.broadcasted_iota(jnp.int32, (D, D), 1)).astype(jnp.float32)
    within = lax.dot_general(eqf, su, (((1,), (0,)), ((), ())),
                             precision=lax.Precision.HIGHEST,
                             preferred_element_type=jnp.float32)
    sl = (lax.broadcasted_iota(jnp.int32, (TR, TR), 1)
          < lax.broadcasted_iota(jnp.int32, (TR, TR), 0)).astype(jnp.float32)
    rowsum = jnp.sum(eqf, axis=1, keepdims=True)
    rowpre = lax.dot_general(sl, rowsum, (((1,), (0,)), ((), ())),
                             precision=lax.Precision.HIGHEST,
                             preferred_element_type=jnp.float32)
    excl = rowpre + within
    needf = (KSEL - jnp.sum(gt.astype(jnp.int32))).astype(jnp.float32)
    u_ref[...] = jnp.where(gt | (eq & (excl < needf)), 1.0, 0.0)


def _topk_mask(score_pad):
    u2d = pl.pallas_call(
        _topk_body,
        out_shape=jax.ShapeDtypeStruct((TR, D), jnp.float32),
    )(score_pad.reshape(TR, D))
    return u2d.reshape(NPAD)


def _mm_body(s_ref, x_ref, w_ref, o_ref):
    xs = s_ref[...] * x_ref[...]
    o_ref[...] = lax.dot_general(
        xs, w_ref[...], (((1,), (0,)), ((), ())),
        precision=lax.Precision.HIGHEST,
        preferred_element_type=jnp.float32)


def _scaled_mm(x, w, s):
    grid = NPAD // BR
    return pl.pallas_call(
        _mm_body,
        grid=(grid,),
        in_specs=[
            pl.BlockSpec((BR, 1), lambda i: (i, 0)),
            pl.BlockSpec((BR, D), lambda i: (i, 0)),
            pl.BlockSpec((D, D), lambda i: (0, 0)),
        ],
        out_specs=pl.BlockSpec((BR, D), lambda i: (i, 0)),
        out_shape=jax.ShapeDtypeStruct((NPAD, D), jnp.float32),
    )(s, x, w)


def _x1_score_body(v1_ref, v2_ref, p0_ref, p1_ref, h_ref, b_ref, pn_ref,
                   x1_ref, sc_ref):
    x1 = jax.nn.relu(v1_ref[...] * (p0_ref[...] + p1_ref[...])
                     + v2_ref[...] * h_ref[...] + b_ref[...])
    x1_ref[...] = x1
    sc_ref[...] = jnp.tanh(
        lax.dot_general(x1, pn_ref[...], (((1,), (0,)), ((), ())),
                        precision=lax.Precision.HIGHEST,
                        preferred_element_type=jnp.float32))


def _x1_score(v1, v2, p0, p1, h0s, b0row, pn):
    grid = NPAD // BR
    vec = pl.BlockSpec((BR, 1), lambda i: (i, 0))
    big = pl.BlockSpec((BR, D), lambda i: (i, 0))
    return pl.pallas_call(
        _x1_score_body,
        grid=(grid,),
        in_specs=[vec, vec, big, big, big,
                  pl.BlockSpec((1, D), lambda i: (0, 0)),
                  pl.BlockSpec((D, 1), lambda i: (0, 0))],
        out_specs=[big, vec],
        out_shape=[jax.ShapeDtypeStruct((NPAD, D), jnp.float32),
                   jax.ShapeDtypeStruct((NPAD, 1), jnp.float32)],
    )(v1, v2, p0, p1, h0s, b0row, pn)


def _t1_body(a_ref, y_ref, t0_ref, t1_ref, o_ref):
    o_ref[...] = a_ref[...] * y_ref[...] + t0_ref[...] + t1_ref[...]


def _t1_combine(a, y, t0, t1):
    grid = NPAD // BR
    vec = pl.BlockSpec((BR, 1), lambda i: (i, 0))
    big = pl.BlockSpec((BR, D), lambda i: (i, 0))
    return pl.pallas_call(
        _t1_body,
        grid=(grid,),
        in_specs=[vec, big, big, big],
        out_specs=big,
        out_shape=jax.ShapeDtypeStruct((NPAD, D), jnp.float32),
    )(a, y, t0, t1)


def _x3_body(a_ref, d2_ref, g_ref, u_ref, b_ref, t1_ref, q0_ref, q1_ref,
             y_ref, x1_ref, o_ref):
    t2 = a_ref[...] * t1_ref[...] + q0_ref[...] + q1_ref[...]
    z = t2 - d2_ref[...] * y_ref[...]
    x2 = jax.nn.relu(g_ref[...] * z + 2.0 * g_ref[...] * y_ref[...]
                     + b_ref[...])
    o_ref[...] = x1_ref[...] + u_ref[...] * x2


def _x3_combine(a, d2v, g1v, uv, b1row, t1, q0, q1, y, x1):
    grid = NPAD // BR
    vec = pl.BlockSpec((BR, 1), lambda i: (i, 0))
    big = pl.BlockSpec((BR, D), lambda i: (i, 0))
    return pl.pallas_call(
        _x3_body,
        grid=(grid,),
        in_specs=[vec, vec, vec, vec,
                  pl.BlockSpec((1, D), lambda i: (0, 0)),
                  big, big, big, big, big],
        out_specs=big,
        out_shape=jax.ShapeDtypeStruct((NPAD, D), jnp.float32),
    )(a, d2v, g1v, uv, b1row, t1, q0, q1, y, x1)


def _out_body(v1_ref, v2_ref, r0_ref, r1_ref, h_ref, b_ref, o_ref):
    o_ref[...] = (v1_ref[...] * (r0_ref[...] + r1_ref[...])
                  + v2_ref[...] * h_ref[...] + b_ref[...])


def _out_combine(v1, v2, r0, r1, hus, burow):
    grid = NPAD // BR
    vec = pl.BlockSpec((BR, 1), lambda i: (i, 0))
    big = pl.BlockSpec((BR, D), lambda i: (i, 0))
    return pl.pallas_call(
        _out_body,
        grid=(grid,),
        in_specs=[vec, vec, big, big, big,
                  pl.BlockSpec((1, D), lambda i: (0, 0))],
        out_specs=big,
        out_shape=jax.ShapeDtypeStruct((NPAD, D), jnp.float32),
    )(v1, v2, r0, r1, hus, burow)


def _pad1(v, fill=0.0):
    return jnp.pad(v, (0, NPAD - N), constant_values=fill).reshape(NPAD, 1)


def kernel(x, edge_index, W0, b0, W1, b1, p, Wu, bu):
    row = edge_index[0].astype(jnp.int32)
    col = edge_index[1].astype(jnp.int32)

    pad_e = NW * EPT - E
    rows_flat = jnp.concatenate([row, jnp.full((pad_e,), PHANTOM, jnp.int32)])
    cols_flat = jnp.concatenate([col, jnp.full((pad_e,), PHANTOM, jnp.int32)])
    rows_sc = rows_flat.reshape(NW, NCHUNK, G)
    cols_sc = cols_flat.reshape(NW, NCHUNK, G)
    zeros_h = jnp.zeros((NPAD, D), jnp.float32)
    zeros2 = jnp.zeros((NPAD, W8), jnp.float32)
    is_self = (row == col).astype(jnp.float32)

    cnt_all = _edge_scat(jnp.ones((E,), jnp.float32), col, zeros2)[:N]
    cnt_self = _edge_scat(is_self, col, zeros2)[:N]
    deg0 = cnt_all + jnp.where(cnt_self > 0, 0.0, 2.0)
    dinv = 1.0 / jnp.sqrt(deg0)

    selfb = row == col
    SENT = jnp.int32(2 * 100000000 + 1)
    vk = (row * N + col) * 2
    vq = jnp.where(selfb, SENT, (col * N + row) * 2 + 1)
    v_all = jnp.concatenate([vk, vq])
    pay = jnp.concatenate(
        [jnp.full((E,), PHANTOM, jnp.int32),
         jnp.where(selfb, PHANTOM, row)])
    v_s, pay_s = lax.sort_key_val(v_all, pay)
    tag = v_s & 1
    cntk = jnp.cumsum(1 - tag)
    keyv = v_s >> 1
    newrun = jnp.concatenate(
        [jnp.array([True]), keyv[1:] != keyv[:-1]])
    prevk = jnp.concatenate([jnp.zeros((1,), cntk.dtype), cntk[:-1]])
    base = jnp.where(newrun, prevk, -1)
    base = lax.cummax(base)
    cnt_q = ((cntk - base) * tag).astype(jnp.float32)
    d2 = 1.0 + _edge_scat(cnt_q, pay_s, zeros2)[:N]

    x_pad = jnp.pad(x, ((0, NPAD - N), (0, 0)))
    loop_w = jnp.where(cnt_self > 0, 0.0, 2.0)
    dinv_p = _pad1(dinv)
    dlw_p = _pad1(dinv * loop_w)
    b0row = b0.reshape(1, D)
    b1row = b1.reshape(1, D)
    burow = bu.reshape(1, D)
    pn = (p / jnp.linalg.norm(p)).reshape(D, 1)

    h0s = _scaled_mm(x_pad, W0, dinv_p)
    p0, p1 = _spmv(h0s, rows_sc, cols_sc, zeros_h)
    x1, score = _x1_score(dinv_p, dlw_p, p0, p1, h0s, b0row, pn)

    u_pad = _topk_mask(score[:, 0])
    u = u_pad[:N]
    score1d = score[:N, 0]

    s1 = (1.0 - cnt_self) * u + _scal_spmv(u_pad, rows_sc, cols_sc,
                                           zeros2)[:N]
    s1_pad = jnp.pad(s1, (0, NPAD - N))
    s2 = (1.0 - cnt_self) * s1 + _scal_spmv(s1_pad, rows_sc, cols_sc,
                                            zeros2)[:N]
    deg1 = s2 - d2 * u + 2.0
    dinv1m = jnp.where(u > 0, 1.0 / jnp.sqrt(deg1), 0.0)

    sc1 = _pad1(dinv1m * score1d)
    y = _scaled_mm(x1, W1, sc1)
    t0a, t0b = _spmv(y, rows_sc, cols_sc, zeros_h)
    a_p = _pad1(1.0 - cnt_self, fill=1.0)
    t1 = _t1_combine(a_p, y, t0a, t0b)
    q0, q1 = _spmv(t1, rows_sc, cols_sc, zeros_h)
    x3 = _x3_combine(a_p, _pad1(d2), _pad1(dinv1m), _pad1(u), b1row,
                     t1, q0, q1, y, x1)

    hus = _scaled_mm(x3, Wu, dinv_p)
    r0, r1 = _spmv(hus, rows_sc, cols_sc, zeros_h)
    out = _out_combine(dinv_p, dlw_p, r0, r1, hus, burow)
    return out[:N]
